# Initial kernel scaffold; baseline (speedup 1.0000x reference)
import jax
import jax.numpy as jnp
from jax.experimental import pallas as pl


def kernel(x, edge_index, split_n, We, be, Wg, bg):
    raise NotImplementedError("write your pallas kernel here")


# speedup vs baseline: 107.1468x; 107.1468x over previous
import functools

import jax
import jax.numpy as jnp
from jax import lax
from jax.experimental import pallas as pl
from jax.experimental.pallas import tpu as pltpu
from jax.experimental.pallas import tpu_sc as plsc

_NC = 2
_NS = 16
_NW = _NC * _NS
_L = 16


def _round_up(v, m):
    return (v + m - 1) // m * m


def _proj_body(x_ref, w_ref, bias_ref, o_ref):
    o_ref[...] = (
        jnp.dot(x_ref[...], w_ref[...], preferred_element_type=jnp.float32)
        + bias_ref[...]
    )


def _prep_body(degp_ref, h_ref, dis_ref, g_ref):
    deg = degp_ref[0, :] + degp_ref[1, :] + 1.0
    dis = lax.rsqrt(deg)
    dis_ref[...] = dis
    g_ref[...] = dis * h_ref[...]


def _soft_body(sp_ref, dis_ref, g_ref, split_ref, bg_ref, nm_ref, ns_ref):
    npad = dis_ref.shape[0]
    nb = split_ref.shape[0]
    spl = split_ref[...]
    ib = lax.broadcasted_iota(jnp.int32, (nb, nb), 0)
    jb = lax.broadcasted_iota(jnp.int32, (nb, nb), 1)
    oincl = jnp.sum(jnp.where(jb <= ib, spl[None, :], 0), axis=1)
    oexcl = oincl - spl
    ii = lax.broadcasted_iota(jnp.int32, (nb, npad), 1)
    mask = (ii >= oexcl[:, None]) & (ii < oincl[:, None])

    def segsoft(v):
        m = jnp.max(jnp.where(mask, v[None, :], -jnp.inf), axis=1)
        mn = jnp.sum(jnp.where(mask, m[:, None], 0.0), axis=0)
        e = jnp.exp(v - mn)
        sb = jnp.sum(jnp.where(mask, e[None, :], 0.0), axis=1)
        sn = jnp.sum(jnp.where(mask, sb[:, None], 0.0), axis=0)
        return e / jnp.maximum(sn, 1e-16)

    s = sp_ref[0, :] + sp_ref[1, :]
    natt = dis_ref[...] * (s + g_ref[...]) + bg_ref[...]
    nm = segsoft(natt)
    nm_ref[...] = nm
    ns_ref[...] = segsoft(1.0 - nm)


def _make_edge_deg_kernel(ep, ew, npad):
    vecs = ew // _L
    nvec = npad // _L
    mesh = plsc.VectorSubcoreMesh(core_axis_name="c", subcore_axis_name="s")

    @functools.partial(
        pl.kernel,
        out_type=[
            jax.ShapeDtypeStruct((ep,), jnp.float32),
            jax.ShapeDtypeStruct((ep,), jnp.float32),
            jax.ShapeDtypeStruct((_NC, npad), jnp.float32),
        ],
        mesh=mesh,
        scratch_types=[
            pltpu.VMEM((ew,), jnp.int32),
            pltpu.VMEM((ew,), jnp.int32),
            pltpu.VMEM((npad,), jnp.float32),
            pltpu.VMEM((npad,), jnp.float32),
            pltpu.VMEM((ew,), jnp.float32),
            pltpu.VMEM((ew,), jnp.float32),
            pltpu.VMEM((ew,), jnp.float32),
            pltpu.VMEM((npad,), jnp.float32),
            pltpu.VMEM_SHARED((npad,), jnp.float32),
        ],
        compiler_params=pltpu.CompilerParams(needs_layout_passes=False),
    )
    def ek(row_h, col_h, a_h, b_h, eam_h, eas_h, degp_h,
           row_v, col_v, a_v, b_v, m_v, s_v, one_v, zero_v, acc):
        c = lax.axis_index("c")
        s = lax.axis_index("s")
        w = s * _NC + c
        base = w * ew
        pltpu.sync_copy(row_h.at[pl.ds(base, ew)], row_v)
        pltpu.sync_copy(col_h.at[pl.ds(base, ew)], col_v)
        pltpu.sync_copy(a_h, a_v)
        pltpu.sync_copy(b_h, b_v)
        ones = jnp.full((_L,), 1.0, jnp.float32)
        zeros = jnp.zeros((_L,), jnp.float32)

        def fill_ones(i, carry):
            one_v[pl.ds(i * _L, _L)] = ones
            return carry

        lax.fori_loop(0, vecs, fill_ones, 0)

        @pl.when(s == 0)
        def _():
            def fill_zeros(i, carry):
                zero_v[pl.ds(i * _L, _L)] = zeros
                return carry

            lax.fori_loop(0, nvec, fill_zeros, 0)
            pltpu.sync_copy(zero_v, acc)

        plsc.subcore_barrier()

        def step(i, carry):
            sl = pl.ds(i * _L, _L)
            av = plsc.load_gather(a_v, [row_v[sl]])
            bv = plsc.load_gather(b_v, [col_v[sl]])
            m = 1.0 / (1.0 + jnp.exp(-(av + bv)))
            m_v[sl] = m
            s_v[sl] = 1.0 - m
            return carry

        lax.fori_loop(0, vecs, step, 0)
        pltpu.sync_copy(one_v, acc.at[row_v], add=True)
        pltpu.sync_copy(m_v, eam_h.at[pl.ds(base, ew)])
        pltpu.sync_copy(s_v, eas_h.at[pl.ds(base, ew)])
        plsc.subcore_barrier()

        @pl.when(s == 0)
        def _():
            pltpu.sync_copy(acc, degp_h.at[c])

    return ek


def _make_scatter_kernel(ep, ew, npad):
    vecs = ew // _L
    nvec = npad // _L
    mesh = plsc.VectorSubcoreMesh(core_axis_name="c", subcore_axis_name="s")

    @functools.partial(
        pl.kernel,
        out_type=[jax.ShapeDtypeStruct((_NC, npad), jnp.float32)],
        mesh=mesh,
        scratch_types=[
            pltpu.VMEM((ew,), jnp.int32),
            pltpu.VMEM((ew,), jnp.int32),
            pltpu.VMEM((npad,), jnp.float32),
            pltpu.VMEM((ew,), jnp.float32),
            pltpu.VMEM((npad,), jnp.float32),
            pltpu.VMEM_SHARED((npad,), jnp.float32),
        ],
        compiler_params=pltpu.CompilerParams(needs_layout_passes=False),
    )
    def sk(row_h, col_h, g_h, sp_h, row_v, col_v, g_v, val_v, zero_v, acc):
        c = lax.axis_index("c")
        s = lax.axis_index("s")
        w = s * _NC + c
        base = w * ew
        pltpu.sync_copy(row_h.at[pl.ds(base, ew)], row_v)
        pltpu.sync_copy(col_h.at[pl.ds(base, ew)], col_v)
        pltpu.sync_copy(g_h, g_v)
        zeros = jnp.zeros((_L,), jnp.float32)

        @pl.when(s == 0)
        def _():
            def fill_zeros(i, carry):
                zero_v[pl.ds(i * _L, _L)] = zeros
                return carry

            lax.fori_loop(0, nvec, fill_zeros, 0)
            pltpu.sync_copy(zero_v, acc)

        plsc.subcore_barrier()

        def step(i, carry):
            sl = pl.ds(i * _L, _L)
            val_v[sl] = plsc.load_gather(g_v, [row_v[sl]])
            return carry

        lax.fori_loop(0, vecs, step, 0)
        pltpu.sync_copy(val_v, acc.at[col_v], add=True)
        plsc.subcore_barrier()

        @pl.when(s == 0)
        def _():
            pltpu.sync_copy(acc, sp_h.at[c])

    return sk


def kernel(x, edge_index, split_n, We, be, Wg, bg):
    n, d = x.shape
    e = edge_index.shape[1]
    nb = split_n.shape[0]
    npad = _round_up(n, _L)
    ew = _round_up(e, _NW * _L) // _NW
    ep = ew * _NW

    xp = jnp.pad(x, ((0, npad - n), (0, 0)))
    w3 = jnp.concatenate([We[:d], We[d:], Wg], axis=1)
    bias = jnp.stack([be[0], jnp.float32(0.0), jnp.float32(0.0)])[None, :]

    proj = pl.pallas_call(
        _proj_body,
        out_shape=jax.ShapeDtypeStruct((npad, 3), jnp.float32),
    )(xp, w3, bias)
    a = proj[:, 0]
    b = proj[:, 1]
    h = proj[:, 2]

    pad_e = jnp.full((ep - e,), n, dtype=jnp.int32)
    rowp = jnp.concatenate([edge_index[0], pad_e])
    colp = jnp.concatenate([edge_index[1], pad_e])

    eam, eas, degp = _make_edge_deg_kernel(ep, ew, npad)(rowp, colp, a, b)

    dis, g = pl.pallas_call(
        _prep_body,
        out_shape=[
            jax.ShapeDtypeStruct((npad,), jnp.float32),
            jax.ShapeDtypeStruct((npad,), jnp.float32),
        ],
    )(degp, h)

    (sp,) = _make_scatter_kernel(ep, ew, npad)(rowp, colp, g)

    nm, ns = pl.pallas_call(
        _soft_body,
        out_shape=[
            jax.ShapeDtypeStruct((npad,), jnp.float32),
            jax.ShapeDtypeStruct((npad,), jnp.float32),
        ],
    )(sp, dis, g, split_n, bg)

    return (eam[:e, None], eas[:e, None], nm[:n, None], ns[:n, None])


# speedup vs baseline: 137.5746x; 1.2840x over previous
import functools

import jax
import jax.numpy as jnp
from jax import lax
from jax.experimental import pallas as pl
from jax.experimental.pallas import tpu as pltpu
from jax.experimental.pallas import tpu_sc as plsc

_NC = 2
_NS = 16
_NW = _NC * _NS
_L = 16


def _round_up(v, m):
    return (v + m - 1) // m * m


def _proj_body(x_ref, w_ref, bias_ref, o_ref):
    o_ref[...] = (
        jnp.dot(x_ref[...], w_ref[...], preferred_element_type=jnp.float32)
        + bias_ref[...]
    )


def _prep_body(degp_ref, h_ref, dis_ref, g_ref):
    deg = degp_ref[0, :] + degp_ref[1, :] + 1.0
    dis = lax.rsqrt(deg)
    dis_ref[...] = dis
    g_ref[...] = dis * h_ref[...]


def _soft_body(sp_ref, dis_ref, g_ref, split_ref, bg_ref, nm_ref, ns_ref):
    npad = dis_ref.shape[0]
    nb = split_ref.shape[0]
    spl = split_ref[...]
    ib = lax.broadcasted_iota(jnp.int32, (nb, nb), 0)
    jb = lax.broadcasted_iota(jnp.int32, (nb, nb), 1)
    oincl = jnp.sum(jnp.where(jb <= ib, spl[None, :], 0), axis=1)
    oexcl = oincl - spl
    ii = lax.broadcasted_iota(jnp.int32, (nb, npad), 1)
    mask = (ii >= oexcl[:, None]) & (ii < oincl[:, None])

    def segsoft(v):
        m = jnp.max(jnp.where(mask, v[None, :], -jnp.inf), axis=1)
        mn = jnp.sum(jnp.where(mask, m[:, None], 0.0), axis=0)
        e = jnp.exp(v - mn)
        sb = jnp.sum(jnp.where(mask, e[None, :], 0.0), axis=1)
        sn = jnp.sum(jnp.where(mask, sb[:, None], 0.0), axis=0)
        return e / jnp.maximum(sn, 1e-16)

    s = sp_ref[0, :] + sp_ref[1, :]
    natt = dis_ref[...] * (s + g_ref[...]) + bg_ref[...]
    nm = segsoft(natt)
    nm_ref[...] = nm
    ns_ref[...] = segsoft(1.0 - nm)


def _make_edge_deg_kernel(ep, ew, npad):
    vecs = ew // _L
    nvec = npad // _L
    mesh = plsc.VectorSubcoreMesh(core_axis_name="c", subcore_axis_name="s")

    @functools.partial(
        pl.kernel,
        out_type=[
            jax.ShapeDtypeStruct((ep,), jnp.float32),
            jax.ShapeDtypeStruct((ep,), jnp.float32),
            jax.ShapeDtypeStruct((_NC, npad), jnp.float32),
        ],
        mesh=mesh,
        scratch_types=[
            pltpu.VMEM((ew,), jnp.int32),
            pltpu.VMEM((ew,), jnp.int32),
            pltpu.VMEM((npad,), jnp.float32),
            pltpu.VMEM((npad,), jnp.float32),
            pltpu.VMEM((ew,), jnp.float32),
            pltpu.VMEM((ew,), jnp.float32),
            pltpu.VMEM((ew,), jnp.float32),
            pltpu.VMEM((npad,), jnp.float32),
            pltpu.VMEM_SHARED((npad,), jnp.float32),
            pltpu.SemaphoreType.DMA,
            pltpu.SemaphoreType.DMA,
            pltpu.SemaphoreType.DMA,
            pltpu.SemaphoreType.DMA,
            pltpu.SemaphoreType.DMA,
            pltpu.SemaphoreType.DMA,
            pltpu.SemaphoreType.DMA,
        ],
        compiler_params=pltpu.CompilerParams(needs_layout_passes=False),
    )
    def ek(row_h, col_h, a_h, b_h, eam_h, eas_h, degp_h,
           row_v, col_v, a_v, b_v, m_v, s_v, one_v, zero_v, acc,
           sem_r, sem_c, sem_a, sem_b, sem_sc, sem_m, sem_s):
        c = lax.axis_index("c")
        s = lax.axis_index("s")
        w = s * _NC + c
        base = w * ew
        dr = pltpu.async_copy(row_h.at[pl.ds(base, ew)], row_v, sem_r)
        dc = pltpu.async_copy(col_h.at[pl.ds(base, ew)], col_v, sem_c)
        da = pltpu.async_copy(a_h, a_v, sem_a)
        db = pltpu.async_copy(b_h, b_v, sem_b)
        ones = jnp.full((_L,), 1.0, jnp.float32)
        zeros = jnp.zeros((_L,), jnp.float32)

        @plsc.parallel_loop(0, vecs, unroll=8)
        def _(i):
            one_v[pl.ds(i * _L, _L)] = ones

        @pl.when(s == 0)
        def _():
            @plsc.parallel_loop(0, nvec, unroll=8)
            def _(i):
                zero_v[pl.ds(i * _L, _L)] = zeros

            pltpu.sync_copy(zero_v, acc)

        plsc.subcore_barrier()
        dr.wait()
        dsc = pltpu.async_copy(one_v, acc.at[row_v], sem_sc, add=True)
        dc.wait()
        da.wait()
        db.wait()

        @plsc.parallel_loop(0, vecs, unroll=4)
        def _(i):
            sl = pl.ds(i * _L, _L)
            av = plsc.load_gather(a_v, [row_v[sl]])
            bv = plsc.load_gather(b_v, [col_v[sl]])
            m = 1.0 / (1.0 + jnp.exp(-(av + bv)))
            m_v[sl] = m
            s_v[sl] = 1.0 - m

        dm = pltpu.async_copy(m_v, eam_h.at[pl.ds(base, ew)], sem_m)
        ds2 = pltpu.async_copy(s_v, eas_h.at[pl.ds(base, ew)], sem_s)
        dsc.wait()
        dm.wait()
        ds2.wait()
        plsc.subcore_barrier()

        @pl.when(s == 0)
        def _():
            pltpu.sync_copy(acc, degp_h.at[c])

    return ek


def _make_scatter_kernel(ep, ew, npad):
    vecs = ew // _L
    nvec = npad // _L
    vlo = (vecs + 1) // 2
    vhi = vecs - vlo
    elo = vlo * _L
    ehi = vhi * _L
    mesh = plsc.VectorSubcoreMesh(core_axis_name="c", subcore_axis_name="s")

    @functools.partial(
        pl.kernel,
        out_type=[jax.ShapeDtypeStruct((_NC, npad), jnp.float32)],
        mesh=mesh,
        scratch_types=[
            pltpu.VMEM((elo,), jnp.int32),
            pltpu.VMEM((ehi,), jnp.int32),
            pltpu.VMEM((elo,), jnp.int32),
            pltpu.VMEM((ehi,), jnp.int32),
            pltpu.VMEM((npad,), jnp.float32),
            pltpu.VMEM((elo,), jnp.float32),
            pltpu.VMEM((ehi,), jnp.float32),
            pltpu.VMEM((npad,), jnp.float32),
            pltpu.VMEM_SHARED((npad,), jnp.float32),
            pltpu.SemaphoreType.DMA,
            pltpu.SemaphoreType.DMA,
            pltpu.SemaphoreType.DMA,
            pltpu.SemaphoreType.DMA,
            pltpu.SemaphoreType.DMA,
            pltpu.SemaphoreType.DMA,
            pltpu.SemaphoreType.DMA,
        ],
        compiler_params=pltpu.CompilerParams(needs_layout_passes=False),
    )
    def sk(row_h, col_h, g_h, sp_h,
           rlo_v, rhi_v, clo_v, chi_v, g_v, vlo_v, vhi_v, zero_v, acc,
           sem_r1, sem_r2, sem_c1, sem_c2, sem_g, sem_lo, sem_hi):
        c = lax.axis_index("c")
        s = lax.axis_index("s")
        w = s * _NC + c
        base = w * ew
        d1 = pltpu.async_copy(row_h.at[pl.ds(base, elo)], rlo_v, sem_r1)
        d2 = pltpu.async_copy(row_h.at[pl.ds(base + elo, ehi)], rhi_v, sem_r2)
        d3 = pltpu.async_copy(col_h.at[pl.ds(base, elo)], clo_v, sem_c1)
        d4 = pltpu.async_copy(col_h.at[pl.ds(base + elo, ehi)], chi_v, sem_c2)
        d5 = pltpu.async_copy(g_h, g_v, sem_g)
        zeros = jnp.zeros((_L,), jnp.float32)

        @pl.when(s == 0)
        def _():
            @plsc.parallel_loop(0, nvec, unroll=8)
            def _(i):
                zero_v[pl.ds(i * _L, _L)] = zeros

            pltpu.sync_copy(zero_v, acc)

        d1.wait()
        d5.wait()
        plsc.subcore_barrier()

        @plsc.parallel_loop(0, vlo, unroll=4)
        def _(i):
            sl = pl.ds(i * _L, _L)
            vlo_v[sl] = plsc.load_gather(g_v, [rlo_v[sl]])

        d3.wait()
        dlo = pltpu.async_copy(vlo_v, acc.at[clo_v], sem_lo, add=True)
        d2.wait()

        @plsc.parallel_loop(0, vhi, unroll=4)
        def _(i):
            sl = pl.ds(i * _L, _L)
            vhi_v[sl] = plsc.load_gather(g_v, [rhi_v[sl]])

        d4.wait()
        dhi = pltpu.async_copy(vhi_v, acc.at[chi_v], sem_hi, add=True)
        dlo.wait()
        dhi.wait()
        plsc.subcore_barrier()

        @pl.when(s == 0)
        def _():
            pltpu.sync_copy(acc, sp_h.at[c])

    return sk


def kernel(x, edge_index, split_n, We, be, Wg, bg):
    n, d = x.shape
    e = edge_index.shape[1]
    nb = split_n.shape[0]
    npad = _round_up(n, _L)
    ew = _round_up(e, _NW * _L) // _NW
    ep = ew * _NW

    xp = jnp.pad(x, ((0, npad - n), (0, 0)))
    w3 = jnp.concatenate([We[:d], We[d:], Wg], axis=1)
    bias = jnp.stack([be[0], jnp.float32(0.0), jnp.float32(0.0)])[None, :]

    proj = pl.pallas_call(
        _proj_body,
        out_shape=jax.ShapeDtypeStruct((npad, 3), jnp.float32),
    )(xp, w3, bias)
    a = proj[:, 0]
    b = proj[:, 1]
    h = proj[:, 2]

    pad_e = jnp.full((ep - e,), n, dtype=jnp.int32)
    rowp = jnp.concatenate([edge_index[0], pad_e])
    colp = jnp.concatenate([edge_index[1], pad_e])

    eam, eas, degp = _make_edge_deg_kernel(ep, ew, npad)(rowp, colp, a, b)

    dis, g = pl.pallas_call(
        _prep_body,
        out_shape=[
            jax.ShapeDtypeStruct((npad,), jnp.float32),
            jax.ShapeDtypeStruct((npad,), jnp.float32),
        ],
    )(degp, h)

    (sp,) = _make_scatter_kernel(ep, ew, npad)(rowp, colp, g)

    nm, ns = pl.pallas_call(
        _soft_body,
        out_shape=[
            jax.ShapeDtypeStruct((npad,), jnp.float32),
            jax.ShapeDtypeStruct((npad,), jnp.float32),
        ],
    )(sp, dis, g, split_n, bg)

    return (eam[:e, None], eas[:e, None], nm[:n, None], ns[:n, None])
